# Initial kernel scaffold; baseline (speedup 1.0000x reference)
import functools

import jax
import jax.numpy as jnp
from jax import lax
from jax.experimental import pallas as pl
from jax.experimental.pallas import tpu as pltpu
from jax.experimental.pallas import tpu_sc as plsc

_NUM_EMBEDDINGS = 1000000
_FEATURES = 64
_TOKENS_SHAPE = (16384, 50)

_B = _TOKENS_SHAPE[0] * _TOKENS_SHAPE[1]
_NC = 2
_NS = 16
_NW = _NC * _NS
_CHUNK = 128
_PER_W = _B // _NW
_NCHUNKS = _PER_W // _CHUNK

_mesh = plsc.VectorSubcoreMesh(core_axis_name="c", subcore_axis_name="s")


@functools.partial(
    pl.kernel,
    mesh=_mesh,
    out_type=jax.ShapeDtypeStruct((_B, _FEATURES), jnp.float32),
    scratch_types=[
        pltpu.VMEM((_NCHUNKS, _CHUNK), jnp.int32),
        pltpu.VMEM((_CHUNK, _FEATURES), jnp.float32),
        pltpu.VMEM((_CHUNK, _FEATURES), jnp.float32),
        pltpu.SemaphoreType.DMA,
        pltpu.SemaphoreType.DMA,
    ],
)
def _embed_sc(tok_hbm, table_hbm, out_hbm, idx_v, rows0, rows1, gsem, ssem):
    wid = lax.axis_index("s") * _NC + lax.axis_index("c")
    base = wid * _PER_W
    pltpu.sync_copy(tok_hbm.at[wid], idx_v)

    def body(j, _):
        cp = pltpu.async_copy(table_hbm.at[idx_v.at[j]], rows0, gsem)
        cp.wait()
        pltpu.sync_copy(rows0, out_hbm.at[pl.ds(base + j * _CHUNK, _CHUNK)])
        return 0

    lax.fori_loop(0, _NCHUNKS, body, 0)


def kernel(tokens, embedding):
    tok = tokens.reshape(-1).astype(jnp.int32).reshape(_NW, _NCHUNKS, _CHUNK)
    out = _embed_sc(tok, embedding)
    return out.reshape(_TOKENS_SHAPE[0], _TOKENS_SHAPE[1], _FEATURES)


# speedup vs baseline: 1.6851x; 1.6851x over previous
import functools

import jax
import jax.numpy as jnp
from jax import lax
from jax.experimental import pallas as pl
from jax.experimental.pallas import tpu as pltpu
from jax.experimental.pallas import tpu_sc as plsc

_NUM_EMBEDDINGS = 1000000
_FEATURES = 64
_TOKENS_SHAPE = (16384, 50)

_B = _TOKENS_SHAPE[0] * _TOKENS_SHAPE[1]
_NC = 2
_NS = 16
_NW = _NC * _NS
_CHUNK = 128
_PER_W = _B // _NW
_NCHUNKS = _PER_W // _CHUNK

_mesh = plsc.VectorSubcoreMesh(core_axis_name="c", subcore_axis_name="s")


@functools.partial(
    pl.kernel,
    mesh=_mesh,
    out_type=jax.ShapeDtypeStruct((_B, _FEATURES), jnp.float32),
    scratch_types=[
        pltpu.VMEM((_NCHUNKS, _CHUNK), jnp.int32),
        pltpu.VMEM((_CHUNK, _FEATURES), jnp.float32),
        pltpu.VMEM((_CHUNK, _FEATURES), jnp.float32),
        pltpu.SemaphoreType.DMA,
        pltpu.SemaphoreType.DMA,
    ],
    compiler_params=pltpu.CompilerParams(use_tc_tiling_on_sc=False),
)
def _embed_sc(tok_hbm, table_hbm, out_hbm, idx_v, rows0, rows1, gsem, ssem):
    wid = lax.axis_index("s") * _NC + lax.axis_index("c")
    base = wid * _PER_W
    pltpu.sync_copy(tok_hbm.at[wid], idx_v)

    def body(j, _):
        cp = pltpu.async_copy(table_hbm.at[idx_v.at[j]], rows0, gsem)
        cp.wait()
        pltpu.sync_copy(rows0, out_hbm.at[pl.ds(base + j * _CHUNK, _CHUNK)])
        return 0

    lax.fori_loop(0, _NCHUNKS, body, 0)


def kernel(tokens, embedding):
    tok = tokens.reshape(-1).astype(jnp.int32).reshape(_NW, _NCHUNKS, _CHUNK)
    out = _embed_sc(tok, embedding)
    return out.reshape(_TOKENS_SHAPE[0], _TOKENS_SHAPE[1], _FEATURES)


# speedup vs baseline: 1.8722x; 1.1110x over previous
import functools

import jax
import jax.numpy as jnp
from jax import lax
from jax.experimental import pallas as pl
from jax.experimental.pallas import tpu as pltpu
from jax.experimental.pallas import tpu_sc as plsc

_NUM_EMBEDDINGS = 1000000
_FEATURES = 64
_TOKENS_SHAPE = (16384, 50)

_B = _TOKENS_SHAPE[0] * _TOKENS_SHAPE[1]
_NC = 2
_NS = 16
_NW = _NC * _NS
_CHUNK = 128
_PER_W = _B // _NW
_NCHUNKS = _PER_W // _CHUNK
_K = 2
_SUPER = _K * _CHUNK
_NSUPER = _NCHUNKS // _K
_NB = 4
_NGROUP = _NSUPER // _NB

_mesh = plsc.VectorSubcoreMesh(core_axis_name="c", subcore_axis_name="s")


@functools.partial(
    pl.kernel,
    mesh=_mesh,
    out_type=jax.ShapeDtypeStruct((_B, _FEATURES), jnp.float32),
    scratch_types=[
        pltpu.VMEM((_NCHUNKS, _CHUNK), jnp.int32),
        [pltpu.VMEM((_SUPER, _FEATURES), jnp.float32) for _ in range(_NB)],
        [pltpu.SemaphoreType.DMA for _ in range(_NB)],
        [pltpu.SemaphoreType.DMA for _ in range(_NB)],
    ],
    compiler_params=pltpu.CompilerParams(use_tc_tiling_on_sc=False),
)
def _embed_sc(tok_hbm, table_hbm, out_hbm, idx_v, bufs, gsems, ssems):
    wid = lax.axis_index("s") * _NC + lax.axis_index("c")
    base = wid * _PER_W
    pltpu.sync_copy(tok_hbm.at[wid], idx_v)

    def fire_gathers(g, b):
        for k in range(_K):
            pltpu.async_copy(
                table_hbm.at[idx_v.at[g * _K + k]],
                bufs[b].at[pl.ds(k * _CHUNK, _CHUNK)],
                gsems[b],
            )

    def drain_gathers(b):
        for k in range(_K):
            pltpu.make_async_copy(
                table_hbm.at[idx_v.at[k]],
                bufs[b].at[pl.ds(k * _CHUNK, _CHUNK)],
                gsems[b],
            ).wait()

    def drain_scatter(b):
        pltpu.make_async_copy(
            bufs[b], out_hbm.at[pl.ds(base, _SUPER)], ssems[b]).wait()

    for b in range(_NB):
        fire_gathers(b, b)

    def body(i, _):
        g0 = i * _NB
        for b in range(_NB):
            drain_gathers(b)
            pltpu.async_copy(
                bufs[b],
                out_hbm.at[pl.ds(base + (g0 + b) * _SUPER, _SUPER)],
                ssems[b],
            )

        @pl.when(i < _NGROUP - 1)
        def _():
            for b in range(_NB):
                drain_scatter(b)
                fire_gathers(g0 + _NB + b, b)

        return 0

    lax.fori_loop(0, _NGROUP, body, 0)
    for b in range(_NB):
        drain_scatter(b)


def kernel(tokens, embedding):
    tok = tokens.reshape(-1).astype(jnp.int32).reshape(_NW, _NCHUNKS, _CHUNK)
    out = _embed_sc(tok, embedding)
    return out.reshape(_TOKENS_SHAPE[0], _TOKENS_SHAPE[1], _FEATURES)
